# baseline (speedup 1.0000x reference)
import functools

import jax
import jax.numpy as jnp
from jax import lax
from jax.experimental import pallas as pl
from jax.experimental.pallas import tpu as pltpu
from jax.experimental.pallas import tpu_sc as plsc

N_NODES_C = 10000
N_EDGES_C = 320000
D_C = 128
NUM_GRAPHS_C = 64

CH = 128
N_CHUNKS = N_EDGES_C // CH
NC = 2
NS = 16
NW = NC * NS
ROW_BLK = 80
N_ROW_BLOCKS = N_NODES_C // ROW_BLK

KMAX = N_CHUNKS // NW + 1


def _sc_body(idx_hbm, edge_hbm, node_hbm, out_hbm,
             ix0, ix1, ix2, b0, b1, b2, agg,
             si0, si1, si2, sd0, sd1, sd2, sc0, sc1, sc2):
    cid = lax.axis_index("c")
    sid = lax.axis_index("s")
    wid = sid * NC + cid

    ix = (ix0, ix1, ix2)
    bf = (b0, b1, b2)
    si = (si0, si1, si2)
    sd = (sd0, sd1, sd2)
    ssc = (sc0, sc1, sc2)
    NSLOT = 3
    NT = (KMAX + NSLOT - 1) // NSLOT

    nk = jnp.where(wid < (N_CHUNKS % NW), KMAX, KMAX - 1)
    base = jnp.where(wid < (N_CHUNKS % NW), wid * KMAX,
                     (N_CHUNKS % NW) + wid * (KMAX - 1))

    zv = jnp.zeros((16,), jnp.float32)

    @pl.loop(0, ROW_BLK)
    def _zero_rows(r):
        for j in range(D_C // 16):
            b0[r, pl.ds(j * 16, 16)] = zv

    @pl.loop(0, (N_ROW_BLOCKS + NS - 1) // NS)
    def _zero_agg(k):
        blk = k * NS + sid

        @pl.when(blk < N_ROW_BLOCKS)
        def _():
            pltpu.sync_copy(
                b0.at[pl.ds(0, ROW_BLK)],
                agg.at[pl.ds(blk * ROW_BLK, ROW_BLK)],
            )

    plsc.subcore_barrier()

    def start_idx(k, s):
        pltpu.async_copy(idx_hbm.at[0, base + k], ix[s].at[0], si[s])
        pltpu.async_copy(idx_hbm.at[1, base + k], ix[s].at[1], si[s])

    def wait_idx(k, s):
        pltpu.make_async_copy(idx_hbm.at[0, base + k], ix[s].at[0], si[s]).wait()
        pltpu.make_async_copy(idx_hbm.at[1, base + k], ix[s].at[1], si[s]).wait()

    def start_edge(k, s):
        pltpu.async_copy(edge_hbm.at[base + k], bf[s], sd[s])

    def wait_edge(k, s):
        pltpu.make_async_copy(edge_hbm.at[base + k], bf[s], sd[s]).wait()

    def start_gather(k, s):
        pltpu.async_copy(node_hbm.at[ix[s].at[0]], bf[s], sd[s])

    def wait_gather(k, s):
        pltpu.make_async_copy(node_hbm.at[ix[s].at[0]], bf[s], sd[s]).wait()

    def scatter(s):
        pltpu.sync_copy(bf[s], agg.at[ix[s].at[1]], add=True)

    def start_gadd(k, s):
        pltpu.async_copy(node_hbm.at[ix[s].at[0]], bf[s], ssc[s], add=True)

    def wait_gadd(k, s):
        pltpu.make_async_copy(node_hbm.at[ix[s].at[0]], bf[s], ssc[s]).wait()

    for s in range(NSLOT):
        start_idx(s, s)
        start_edge(s, s)
    wait_edge(0, 0)
    wait_idx(0, 0)
    start_gadd(0, 0)

    @pl.loop(0, NT)
    def _pipe(t):
        for s in range(NSLOT):
            k = NSLOT * t + s
            s1 = (s + 1) % NSLOT

            @pl.when(k < nk)
            def _():
                wait_gadd(k, s)
                scatter(s)

                @pl.when(k + NSLOT < nk)
                def _():
                    start_idx(k + NSLOT, s)
                    start_edge(k + NSLOT, s)

                @pl.when(k + 1 < nk)
                def _():
                    wait_edge(k + 1, s1)
                    wait_idx(k + 1, s1)
                    start_gadd(k + 1, s1)

    plsc.subcore_barrier()

    @pl.loop(0, (N_ROW_BLOCKS + NS - 1) // NS)
    def _copy_out(k):
        blk = k * NS + sid

        @pl.when(blk < N_ROW_BLOCKS)
        def _():
            pltpu.sync_copy(
                agg.at[pl.ds(blk * ROW_BLK, ROW_BLK)],
                out_hbm.at[cid, pl.ds(blk * ROW_BLK, ROW_BLK)],
            )


@functools.cache
def _sc_aggregate():
    mesh = plsc.VectorSubcoreMesh(
        core_axis_name="c", subcore_axis_name="s",
        num_cores=NC, num_subcores=NS)
    return pl.kernel(
        _sc_body,
        out_type=jax.ShapeDtypeStruct((NC, N_NODES_C, D_C), jnp.float32),
        mesh=mesh,
        scratch_types=[
            pltpu.VMEM((2, CH), jnp.int32),
            pltpu.VMEM((2, CH), jnp.int32),
            pltpu.VMEM((2, CH), jnp.int32),
            pltpu.VMEM((CH, D_C), jnp.float32),
            pltpu.VMEM((CH, D_C), jnp.float32),
            pltpu.VMEM((CH, D_C), jnp.float32),
            pltpu.VMEM_SHARED((N_NODES_C, D_C), jnp.float32),
            pltpu.SemaphoreType.DMA,
            pltpu.SemaphoreType.DMA,
            pltpu.SemaphoreType.DMA,
            pltpu.SemaphoreType.DMA,
            pltpu.SemaphoreType.DMA,
            pltpu.SemaphoreType.DMA,
            pltpu.SemaphoreType.DMA,
            pltpu.SemaphoreType.DMA,
            pltpu.SemaphoreType.DMA,
        ],
    )


BLK = 2000


def _tc_body(agg_ref, node_ref, batch_ref, w1_ref, b1_ref, w2_ref, b2_ref,
             eps_ref, g_ref, bt_ref, out_ref, scale_ref):
    i = pl.program_id(0)

    @pl.when(i == 0)
    def _():
        bfull = batch_ref[...]
        giota = lax.broadcasted_iota(jnp.int32, (N_NODES_C, NUM_GRAPHS_C), 1)
        onehot = (bfull == giota).astype(jnp.float32)
        counts = jnp.sum(onehot, axis=0, keepdims=True)
        scale_ref[...] = lax.rsqrt(jnp.maximum(counts, 1.0))

    aggs = agg_ref[...]
    node = node_ref[...]
    x = aggs[0] + aggs[1] + (1.0 + eps_ref[0, 0]) * node
    h = jnp.dot(x, w1_ref[...], preferred_element_type=jnp.float32,
                precision=lax.Precision.HIGHEST) + b1_ref[...]
    h = jnp.maximum(h, 0.0)
    y = jnp.dot(h, w2_ref[...], preferred_element_type=jnp.float32,
                precision=lax.Precision.HIGHEST) + b2_ref[...]
    mu = jnp.mean(y, axis=-1, keepdims=True)
    var = jnp.mean((y - mu) ** 2, axis=-1, keepdims=True)
    y = (y - mu) * lax.rsqrt(var + 1e-5) * g_ref[...] + bt_ref[...]
    scale_g = scale_ref[...]
    bblk = batch_ref[pl.ds(i * BLK, BLK), :]
    biota = lax.broadcasted_iota(jnp.int32, (BLK, NUM_GRAPHS_C), 1)
    mask = (bblk == biota).astype(jnp.float32)
    scale = jnp.sum(mask * scale_g, axis=1, keepdims=True)
    y = jnp.maximum(y * scale, 0.0)
    out_ref[...] = y + node


_tc_mlp = pl.pallas_call(
    _tc_body,
    grid=(N_NODES_C // BLK,),
    in_specs=[
        pl.BlockSpec((NC, BLK, D_C), lambda i: (0, i, 0)),
        pl.BlockSpec((BLK, D_C), lambda i: (i, 0)),
        pl.BlockSpec((N_NODES_C, 1), lambda i: (0, 0)),
        pl.BlockSpec((D_C, 2 * D_C), lambda i: (0, 0)),
        pl.BlockSpec((1, 2 * D_C), lambda i: (0, 0)),
        pl.BlockSpec((2 * D_C, D_C), lambda i: (0, 0)),
        pl.BlockSpec((1, D_C), lambda i: (0, 0)),
        pl.BlockSpec((1, 1), lambda i: (0, 0)),
        pl.BlockSpec((1, D_C), lambda i: (0, 0)),
        pl.BlockSpec((1, D_C), lambda i: (0, 0)),
    ],
    out_specs=pl.BlockSpec((BLK, D_C), lambda i: (i, 0)),
    out_shape=jax.ShapeDtypeStruct((N_NODES_C, D_C), jnp.float32),
    scratch_shapes=[pltpu.VMEM((1, NUM_GRAPHS_C), jnp.float32)],
)


def kernel(node_hidden, edge_index, edge_hidden, batch, W1, b1, W2, b2, eps,
           ln_gamma, ln_beta):
    idx2 = edge_index.astype(jnp.int32).reshape(2, N_CHUNKS, CH)
    edge3 = edge_hidden.reshape(N_CHUNKS, CH, D_C)
    agg2 = _sc_aggregate()(idx2, edge3, node_hidden)
    out = _tc_mlp(
        agg2,
        node_hidden,
        batch.astype(jnp.int32).reshape(N_NODES_C, 1),
        W1,
        b1.reshape(1, 2 * D_C),
        W2,
        b2.reshape(1, D_C),
        eps.reshape(1, 1),
        ln_gamma.reshape(1, D_C),
        ln_beta.reshape(1, D_C),
    )
    return out


# speedup vs baseline: 1.0583x; 1.0583x over previous
import functools

import jax
import jax.numpy as jnp
from jax import lax
from jax.experimental import pallas as pl
from jax.experimental.pallas import tpu as pltpu
from jax.experimental.pallas import tpu_sc as plsc

N_NODES_C = 10000
N_EDGES_C = 320000
D_C = 128
NUM_GRAPHS_C = 64

CH = 80
N_CHUNKS = N_EDGES_C // CH
NC = 2
NS = 16
NW = NC * NS
NK = N_CHUNKS // NW
ROW_BLK = 80
N_ROW_BLOCKS = N_NODES_C // ROW_BLK

NB = 4
NI = 8
NSEC = 8
NT = (NK + NSEC) // NSEC


def _sc_body(idx_hbm, edge_hbm, node_hbm, out_hbm,
             ix0, ix1, ix2, ix3, ix4, ix5, ix6, ix7,
             b0, b1, b2, b3, agg,
             si0, si1, si2, si3, si4, si5, si6, si7,
             sd0, sd1, sd2, sd3, sc0, sc1, sc2, sc3):
    cid = lax.axis_index("c")
    sid = lax.axis_index("s")
    wid = sid * NC + cid

    ix = (ix0, ix1, ix2, ix3, ix4, ix5, ix6, ix7)
    bf = (b0, b1, b2, b3)
    si = (si0, si1, si2, si3, si4, si5, si6, si7)
    sd = (sd0, sd1, sd2, sd3)
    ssc = (sc0, sc1, sc2, sc3)

    base = wid * NK

    zv = jnp.zeros((16,), jnp.float32)

    @pl.loop(0, ROW_BLK)
    def _zero_rows(r):
        for j in range(D_C // 16):
            b0[r, pl.ds(j * 16, 16)] = zv

    @pl.loop(0, (N_ROW_BLOCKS + NS - 1) // NS)
    def _zero_agg(k):
        blk = k * NS + sid

        @pl.when(blk < N_ROW_BLOCKS)
        def _():
            pltpu.sync_copy(
                b0.at[pl.ds(0, ROW_BLK)],
                agg.at[pl.ds(blk * ROW_BLK, ROW_BLK)],
            )

    plsc.subcore_barrier()

    def start_idx(k, s):
        pltpu.async_copy(idx_hbm.at[0, base + k], ix[s].at[0], si[s])
        pltpu.async_copy(idx_hbm.at[1, base + k], ix[s].at[1], si[s])

    def wait_idx(k, s):
        pltpu.make_async_copy(idx_hbm.at[0, base + k], ix[s].at[0], si[s]).wait()
        pltpu.make_async_copy(idx_hbm.at[1, base + k], ix[s].at[1], si[s]).wait()

    def start_edge(k, s):
        pltpu.async_copy(edge_hbm.at[base + k], bf[s], sd[s])

    def wait_edge(k, s):
        pltpu.make_async_copy(edge_hbm.at[base + k], bf[s], sd[s]).wait()

    def start_gather(k, bs, isl):
        pltpu.async_copy(node_hbm.at[ix[isl].at[0]], bf[bs], sd[bs])

    def wait_gather(k, bs, isl):
        pltpu.make_async_copy(node_hbm.at[ix[isl].at[0]], bf[bs], sd[bs]).wait()

    def start_scat(bs, isl):
        pltpu.async_copy(bf[bs], agg.at[ix[isl].at[1]], ssc[bs], add=True)

    def wait_scat(bs, isl):
        pltpu.make_async_copy(bf[bs], agg.at[ix[isl].at[1]], ssc[bs]).wait()

    def drain_scats():
        wait_scat((NK - 2) % NB, (NK - 2) % NI)
        wait_scat((NK - 1) % NB, (NK - 1) % NI)

    for j in range(NB):
        start_idx(j, j)
    for j in range(2):
        start_edge(j, j)

    @pl.loop(0, NT)
    def _pass_a(u):
        for j in range(NSEC):
            k = NSEC * u + j
            bs = j % NB
            bs2 = (j + 2) % NB
            is4 = (j + 4) % NI

            @pl.when(k < NK)
            def _():
                wait_edge(k, bs)
                wait_idx(k, j)
                start_scat(bs, j)

                @pl.when(k + 2 < NK)
                def _():
                    if j >= 2:
                        wait_scat(bs2, (j - 2) % NI)
                    else:
                        @pl.when(k >= 2)
                        def _():
                            wait_scat(bs2, (j - 2) % NI)

                    start_edge(k + 2, bs2)

                @pl.when(k + 4 < NK)
                def _():
                    start_idx(k + 4, is4)

    drain_scats()

    for j in range(NB):
        start_idx(j, j)
    for j in range(2):
        wait_idx(j, j)
        start_gather(j, j, j)

    @pl.loop(0, NT)
    def _pass_b(u):
        for j in range(NSEC):
            k = NSEC * u + j
            bs = j % NB
            bs2 = (j + 2) % NB
            is2 = (j + 2) % NI
            is4 = (j + 4) % NI

            @pl.when(k < NK)
            def _():
                wait_gather(k, bs, j)
                start_scat(bs, j)

                @pl.when(k + 2 < NK)
                def _():
                    if j >= 2:
                        wait_scat(bs2, (j - 2) % NI)
                    else:
                        @pl.when(k >= 2)
                        def _():
                            wait_scat(bs2, (j - 2) % NI)

                    wait_idx(k + 2, is2)
                    start_gather(k + 2, bs2, is2)

                @pl.when(k + 4 < NK)
                def _():
                    start_idx(k + 4, is4)

    drain_scats()

    plsc.subcore_barrier()

    @pl.loop(0, (N_ROW_BLOCKS + NS - 1) // NS)
    def _copy_out(k):
        blk = k * NS + sid

        @pl.when(blk < N_ROW_BLOCKS)
        def _():
            pltpu.sync_copy(
                agg.at[pl.ds(blk * ROW_BLK, ROW_BLK)],
                out_hbm.at[cid, pl.ds(blk * ROW_BLK, ROW_BLK)],
            )


@functools.cache
def _sc_aggregate():
    mesh = plsc.VectorSubcoreMesh(
        core_axis_name="c", subcore_axis_name="s",
        num_cores=NC, num_subcores=NS)
    return pl.kernel(
        _sc_body,
        out_type=jax.ShapeDtypeStruct((NC, N_NODES_C, D_C), jnp.float32),
        mesh=mesh,
        scratch_types=(
            [pltpu.VMEM((2, CH), jnp.int32) for _ in range(NI)]
            + [pltpu.VMEM((CH, D_C), jnp.float32) for _ in range(NB)]
            + [pltpu.VMEM_SHARED((N_NODES_C, D_C), jnp.float32)]
            + [pltpu.SemaphoreType.DMA for _ in range(NI + 2 * NB)]
        ),
    )


BLK = 2000


def _tc_body(agg_ref, node_ref, batch_ref, w1_ref, b1_ref, w2_ref, b2_ref,
             eps_ref, g_ref, bt_ref, out_ref, scale_ref):
    i = pl.program_id(0)

    @pl.when(i == 0)
    def _():
        bfull = batch_ref[...]
        giota = lax.broadcasted_iota(jnp.int32, (N_NODES_C, NUM_GRAPHS_C), 1)
        onehot = (bfull == giota).astype(jnp.float32)
        counts = jnp.sum(onehot, axis=0, keepdims=True)
        scale_ref[...] = lax.rsqrt(jnp.maximum(counts, 1.0))

    aggs = agg_ref[...]
    node = node_ref[...]
    x = aggs[0] + aggs[1] + (1.0 + eps_ref[0, 0]) * node
    h = jnp.dot(x, w1_ref[...], preferred_element_type=jnp.float32,
                precision=lax.Precision.HIGHEST) + b1_ref[...]
    h = jnp.maximum(h, 0.0)
    y = jnp.dot(h, w2_ref[...], preferred_element_type=jnp.float32,
                precision=lax.Precision.HIGHEST) + b2_ref[...]
    mu = jnp.mean(y, axis=-1, keepdims=True)
    var = jnp.mean((y - mu) ** 2, axis=-1, keepdims=True)
    y = (y - mu) * lax.rsqrt(var + 1e-5) * g_ref[...] + bt_ref[...]
    scale_g = scale_ref[...]
    bblk = batch_ref[pl.ds(i * BLK, BLK), :]
    biota = lax.broadcasted_iota(jnp.int32, (BLK, NUM_GRAPHS_C), 1)
    mask = (bblk == biota).astype(jnp.float32)
    scale = jnp.sum(mask * scale_g, axis=1, keepdims=True)
    y = jnp.maximum(y * scale, 0.0)
    out_ref[...] = y + node


_tc_mlp = pl.pallas_call(
    _tc_body,
    grid=(N_NODES_C // BLK,),
    in_specs=[
        pl.BlockSpec((NC, BLK, D_C), lambda i: (0, i, 0)),
        pl.BlockSpec((BLK, D_C), lambda i: (i, 0)),
        pl.BlockSpec((N_NODES_C, 1), lambda i: (0, 0)),
        pl.BlockSpec((D_C, 2 * D_C), lambda i: (0, 0)),
        pl.BlockSpec((1, 2 * D_C), lambda i: (0, 0)),
        pl.BlockSpec((2 * D_C, D_C), lambda i: (0, 0)),
        pl.BlockSpec((1, D_C), lambda i: (0, 0)),
        pl.BlockSpec((1, 1), lambda i: (0, 0)),
        pl.BlockSpec((1, D_C), lambda i: (0, 0)),
        pl.BlockSpec((1, D_C), lambda i: (0, 0)),
    ],
    out_specs=pl.BlockSpec((BLK, D_C), lambda i: (i, 0)),
    out_shape=jax.ShapeDtypeStruct((N_NODES_C, D_C), jnp.float32),
    scratch_shapes=[pltpu.VMEM((1, NUM_GRAPHS_C), jnp.float32)],
)


def kernel(node_hidden, edge_index, edge_hidden, batch, W1, b1, W2, b2, eps,
           ln_gamma, ln_beta):
    idx2 = edge_index.astype(jnp.int32).reshape(2, N_CHUNKS, CH)
    edge3 = edge_hidden.reshape(N_CHUNKS, CH, D_C)
    agg2 = _sc_aggregate()(idx2, edge3, node_hidden)
    out = _tc_mlp(
        agg2,
        node_hidden,
        batch.astype(jnp.int32).reshape(N_NODES_C, 1),
        W1,
        b1.reshape(1, 2 * D_C),
        W2,
        b2.reshape(1, D_C),
        eps.reshape(1, 1),
        ln_gamma.reshape(1, D_C),
        ln_beta.reshape(1, D_C),
    )
    return out


# speedup vs baseline: 1.2004x; 1.1343x over previous
import functools

import jax
import jax.numpy as jnp
from jax import lax
from jax.experimental import pallas as pl
from jax.experimental.pallas import tpu as pltpu
from jax.experimental.pallas import tpu_sc as plsc

N_NODES_C = 10000
N_EDGES_C = 320000
D_C = 128
NUM_GRAPHS_C = 64

CH = 128
N_CHUNKS = N_EDGES_C // CH
NC = 2
NS = 16
NW = NC * NS
ROW_BLK = 80
N_ROW_BLOCKS = N_NODES_C // ROW_BLK

KMAX = N_CHUNKS // NW + 1


def _sc_body(idx_hbm, edge_hbm, node_hbm, out_hbm,
             ix0, ix1, ix2, b0, b1, b2, agg,
             si0, si1, si2, sd0, sd1, sd2, sc0, sc1, sc2):
    cid = lax.axis_index("c")
    sid = lax.axis_index("s")
    wid = sid * NC + cid

    ix = (ix0, ix1, ix2)
    bf = (b0, b1, b2)
    si = (si0, si1, si2)
    sd = (sd0, sd1, sd2)
    ssc = (sc0, sc1, sc2)
    NSLOT = 3
    NT = (KMAX + NSLOT - 1) // NSLOT

    nk = jnp.where(wid < (N_CHUNKS % NW), KMAX, KMAX - 1)
    base = jnp.where(wid < (N_CHUNKS % NW), wid * KMAX,
                     (N_CHUNKS % NW) + wid * (KMAX - 1))

    def start_idx(k, s):
        pltpu.async_copy(idx_hbm.at[0, base + k], ix[s].at[0], si[s])
        pltpu.async_copy(idx_hbm.at[1, base + k], ix[s].at[1], si[s])

    def wait_idx(k, s):
        pltpu.make_async_copy(idx_hbm.at[0, base + k], ix[s].at[0], si[s]).wait()
        pltpu.make_async_copy(idx_hbm.at[1, base + k], ix[s].at[1], si[s]).wait()

    def start_idx_dst(k, s):
        pltpu.async_copy(idx_hbm.at[1, base + k], ix[s].at[1], si[s])

    def wait_idx_dst(k, s):
        pltpu.make_async_copy(idx_hbm.at[1, base + k], ix[s].at[1], si[s]).wait()

    def start_edge(k, s):
        pltpu.async_copy(edge_hbm.at[base + k], bf[s], sd[s])

    def wait_edge(k, s):
        pltpu.make_async_copy(edge_hbm.at[base + k], bf[s], sd[s]).wait()

    def start_gather(k, s):
        pltpu.async_copy(node_hbm.at[ix[s].at[0]], bf[s], sd[s])

    def wait_gather(k, s):
        pltpu.make_async_copy(node_hbm.at[ix[s].at[0]], bf[s], sd[s]).wait()

    def scatter(s):
        pltpu.sync_copy(bf[s], agg.at[ix[s].at[1]], add=True)

    def run_pass_a():
        for s in range(NSLOT):
            start_idx_dst(s, s)
        start_edge(0, 0)
        start_edge(1, 1)

        zv = jnp.zeros((16,), jnp.float32)

        @pl.loop(0, ROW_BLK)
        def _zero_rows(r):
            for j in range(D_C // 16):
                b2[r, pl.ds(j * 16, 16)] = zv

        @pl.loop(0, (N_ROW_BLOCKS + NS - 1) // NS)
        def _zero_agg(kz):
            blk = kz * NS + sid

            @pl.when(blk < N_ROW_BLOCKS)
            def _():
                pltpu.sync_copy(
                    b2.at[pl.ds(0, ROW_BLK)],
                    agg.at[pl.ds(blk * ROW_BLK, ROW_BLK)],
                )

        plsc.subcore_barrier()
        start_edge(2, 2)

        @pl.loop(0, NT)
        def _pass_a(t):
            for s in range(NSLOT):
                k = NSLOT * t + s

                @pl.when(k < nk)
                def _():
                    wait_edge(k, s)
                    wait_idx_dst(k, s)
                    scatter(s)

                    @pl.when(k + NSLOT < nk)
                    def _():
                        start_idx_dst(k + NSLOT, s)
                        start_edge(k + NSLOT, s)

    def run_pass_b():
        for s in range(NSLOT):
            start_idx(s, s)
        for s in range(NSLOT - 1):
            wait_idx(s, s)
            start_gather(s, s)

        @pl.loop(0, NT)
        def _pass_b(t):
            for s in range(NSLOT):
                k = NSLOT * t + s
                s2 = (s + 2) % NSLOT

                @pl.when(k < nk)
                def _():
                    wait_gather(k, s)
                    scatter(s)

                    @pl.when(k + NSLOT < nk)
                    def _():
                        start_idx(k + NSLOT, s)

                    @pl.when(k + 2 < nk)
                    def _():
                        wait_idx(k + 2, s2)
                        start_gather(k + 2, s2)

    run_pass_a()
    run_pass_b()

    plsc.subcore_barrier()

    @pl.loop(0, (N_ROW_BLOCKS + NS - 1) // NS)
    def _copy_out(k):
        blk = k * NS + sid

        @pl.when(blk < N_ROW_BLOCKS)
        def _():
            pltpu.sync_copy(
                agg.at[pl.ds(blk * ROW_BLK, ROW_BLK)],
                out_hbm.at[cid, pl.ds(blk * ROW_BLK, ROW_BLK)],
            )


@functools.cache
def _sc_aggregate():
    mesh = plsc.VectorSubcoreMesh(
        core_axis_name="c", subcore_axis_name="s",
        num_cores=NC, num_subcores=NS)
    return pl.kernel(
        _sc_body,
        out_type=jax.ShapeDtypeStruct((NC, N_NODES_C, D_C), jnp.float32),
        mesh=mesh,
        scratch_types=[
            pltpu.VMEM((2, CH), jnp.int32),
            pltpu.VMEM((2, CH), jnp.int32),
            pltpu.VMEM((2, CH), jnp.int32),
            pltpu.VMEM((CH, D_C), jnp.float32),
            pltpu.VMEM((CH, D_C), jnp.float32),
            pltpu.VMEM((CH, D_C), jnp.float32),
            pltpu.VMEM_SHARED((N_NODES_C, D_C), jnp.float32),
            pltpu.SemaphoreType.DMA,
            pltpu.SemaphoreType.DMA,
            pltpu.SemaphoreType.DMA,
            pltpu.SemaphoreType.DMA,
            pltpu.SemaphoreType.DMA,
            pltpu.SemaphoreType.DMA,
            pltpu.SemaphoreType.DMA,
            pltpu.SemaphoreType.DMA,
            pltpu.SemaphoreType.DMA,
        ],
    )


BLK = 2000


def _tc_body(agg_ref, node_ref, batch_ref, w1_ref, b1_ref, w2_ref, b2_ref,
             eps_ref, g_ref, bt_ref, out_ref, scale_ref):
    i = pl.program_id(0)

    @pl.when(i == 0)
    def _():
        bfull = batch_ref[...]
        giota = lax.broadcasted_iota(jnp.int32, (N_NODES_C, NUM_GRAPHS_C), 1)
        onehot = (bfull == giota).astype(jnp.float32)
        counts = jnp.sum(onehot, axis=0, keepdims=True)
        scale_ref[...] = lax.rsqrt(jnp.maximum(counts, 1.0))

    aggs = agg_ref[...]
    node = node_ref[...]
    x = aggs[0] + aggs[1] + (1.0 + eps_ref[0, 0]) * node
    h = jnp.dot(x, w1_ref[...], preferred_element_type=jnp.float32,
                precision=lax.Precision.HIGHEST) + b1_ref[...]
    h = jnp.maximum(h, 0.0)
    y = jnp.dot(h, w2_ref[...], preferred_element_type=jnp.float32,
                precision=lax.Precision.HIGHEST) + b2_ref[...]
    mu = jnp.mean(y, axis=-1, keepdims=True)
    var = jnp.mean((y - mu) ** 2, axis=-1, keepdims=True)
    y = (y - mu) * lax.rsqrt(var + 1e-5) * g_ref[...] + bt_ref[...]
    scale_g = scale_ref[...]
    bblk = batch_ref[pl.ds(i * BLK, BLK), :]
    biota = lax.broadcasted_iota(jnp.int32, (BLK, NUM_GRAPHS_C), 1)
    mask = (bblk == biota).astype(jnp.float32)
    scale = jnp.sum(mask * scale_g, axis=1, keepdims=True)
    y = jnp.maximum(y * scale, 0.0)
    out_ref[...] = y + node


_tc_mlp = pl.pallas_call(
    _tc_body,
    grid=(N_NODES_C // BLK,),
    in_specs=[
        pl.BlockSpec((NC, BLK, D_C), lambda i: (0, i, 0)),
        pl.BlockSpec((BLK, D_C), lambda i: (i, 0)),
        pl.BlockSpec((N_NODES_C, 1), lambda i: (0, 0)),
        pl.BlockSpec((D_C, 2 * D_C), lambda i: (0, 0)),
        pl.BlockSpec((1, 2 * D_C), lambda i: (0, 0)),
        pl.BlockSpec((2 * D_C, D_C), lambda i: (0, 0)),
        pl.BlockSpec((1, D_C), lambda i: (0, 0)),
        pl.BlockSpec((1, 1), lambda i: (0, 0)),
        pl.BlockSpec((1, D_C), lambda i: (0, 0)),
        pl.BlockSpec((1, D_C), lambda i: (0, 0)),
    ],
    out_specs=pl.BlockSpec((BLK, D_C), lambda i: (i, 0)),
    out_shape=jax.ShapeDtypeStruct((N_NODES_C, D_C), jnp.float32),
    scratch_shapes=[pltpu.VMEM((1, NUM_GRAPHS_C), jnp.float32)],
)


def kernel(node_hidden, edge_index, edge_hidden, batch, W1, b1, W2, b2, eps,
           ln_gamma, ln_beta):
    idx2 = edge_index.astype(jnp.int32).reshape(2, N_CHUNKS, CH)
    edge3 = edge_hidden.reshape(N_CHUNKS, CH, D_C)
    agg2 = _sc_aggregate()(idx2, edge3, node_hidden)
    out = _tc_mlp(
        agg2,
        node_hidden,
        batch.astype(jnp.int32).reshape(N_NODES_C, 1),
        W1,
        b1.reshape(1, 2 * D_C),
        W2,
        b2.reshape(1, D_C),
        eps.reshape(1, 1),
        ln_gamma.reshape(1, D_C),
        ln_beta.reshape(1, D_C),
    )
    return out


# speedup vs baseline: 1.2129x; 1.0104x over previous
import functools

import jax
import jax.numpy as jnp
from jax import lax
from jax.experimental import pallas as pl
from jax.experimental.pallas import tpu as pltpu
from jax.experimental.pallas import tpu_sc as plsc

N_NODES_C = 10000
N_EDGES_C = 320000
D_C = 128
NUM_GRAPHS_C = 64

CH = 128
N_CHUNKS = N_EDGES_C // CH
NC = 2
NS = 16
NW = NC * NS
ROW_BLK = 80
N_ROW_BLOCKS = N_NODES_C // ROW_BLK

KMAX = N_CHUNKS // NW + 1


def _sc_body(idx_hbm, edge_hbm, node_hbm, out_hbm,
             ix0, ix1, ix2, b0, b1, b2, agg,
             si0, si1, si2, sd0, sd1, sd2, sc0, sc1, sc2):
    cid = lax.axis_index("c")
    sid = lax.axis_index("s")
    wid = sid * NC + cid

    ix = (ix0, ix1, ix2)
    bf = (b0, b1, b2)
    si = (si0, si1, si2)
    sd = (sd0, sd1, sd2)
    ssc = (sc0, sc1, sc2)
    NSLOT = 3
    NT = (KMAX + NSLOT - 1) // NSLOT

    nk = jnp.where(wid < (N_CHUNKS % NW), KMAX, KMAX - 1)
    base = jnp.where(wid < (N_CHUNKS % NW), wid * KMAX,
                     (N_CHUNKS % NW) + wid * (KMAX - 1))

    def start_idx(k, s):
        off = (base + k) * CH
        pltpu.async_copy(idx_hbm.at[0, pl.ds(off, CH)], ix[s].at[0], si[s])
        pltpu.async_copy(idx_hbm.at[1, pl.ds(off, CH)], ix[s].at[1], si[s])

    def wait_idx(k, s):
        off = (base + k) * CH
        pltpu.make_async_copy(idx_hbm.at[0, pl.ds(off, CH)], ix[s].at[0],
                              si[s]).wait()
        pltpu.make_async_copy(idx_hbm.at[1, pl.ds(off, CH)], ix[s].at[1],
                              si[s]).wait()

    def start_idx_dst(k, s):
        pltpu.async_copy(idx_hbm.at[1, pl.ds((base + k) * CH, CH)],
                         ix[s].at[1], si[s])

    def wait_idx_dst(k, s):
        pltpu.make_async_copy(idx_hbm.at[1, pl.ds((base + k) * CH, CH)],
                              ix[s].at[1], si[s]).wait()

    def start_edge(k, s):
        pltpu.async_copy(edge_hbm.at[base + k], bf[s], sd[s])

    def wait_edge(k, s):
        pltpu.make_async_copy(edge_hbm.at[base + k], bf[s], sd[s]).wait()

    def start_gather(k, s):
        pltpu.async_copy(node_hbm.at[ix[s].at[0]], bf[s], sd[s])

    def wait_gather(k, s):
        pltpu.make_async_copy(node_hbm.at[ix[s].at[0]], bf[s], sd[s]).wait()

    def scatter(s):
        pltpu.sync_copy(bf[s], agg.at[ix[s].at[1]], add=True)

    def run_pass_a():
        for s in range(NSLOT):
            start_idx_dst(s, s)
        start_edge(0, 0)
        start_edge(1, 1)

        zv = jnp.zeros((16,), jnp.float32)

        @pl.loop(0, ROW_BLK)
        def _zero_rows(r):
            for j in range(D_C // 16):
                b2[r, pl.ds(j * 16, 16)] = zv

        @pl.loop(0, (N_ROW_BLOCKS + NS - 1) // NS)
        def _zero_agg(kz):
            blk = kz * NS + sid

            @pl.when(blk < N_ROW_BLOCKS)
            def _():
                pltpu.sync_copy(
                    b2.at[pl.ds(0, ROW_BLK)],
                    agg.at[pl.ds(blk * ROW_BLK, ROW_BLK)],
                )

        plsc.subcore_barrier()
        start_edge(2, 2)

        @pl.loop(0, NT)
        def _pass_a(t):
            for s in range(NSLOT):
                k = NSLOT * t + s

                @pl.when(k < nk)
                def _():
                    wait_edge(k, s)
                    wait_idx_dst(k, s)
                    scatter(s)

                    @pl.when(k + NSLOT < nk)
                    def _():
                        start_idx_dst(k + NSLOT, s)
                        start_edge(k + NSLOT, s)

    def run_pass_b():
        for s in range(NSLOT):
            start_idx(s, s)
        for s in range(NSLOT - 1):
            wait_idx(s, s)
            start_gather(s, s)

        @pl.loop(0, NT)
        def _pass_b(t):
            for s in range(NSLOT):
                k = NSLOT * t + s
                s2 = (s + 2) % NSLOT

                @pl.when(k < nk)
                def _():
                    wait_gather(k, s)
                    scatter(s)

                    @pl.when(k + NSLOT < nk)
                    def _():
                        start_idx(k + NSLOT, s)

                    @pl.when(k + 2 < nk)
                    def _():
                        wait_idx(k + 2, s2)
                        start_gather(k + 2, s2)

    run_pass_a()
    run_pass_b()

    plsc.subcore_barrier()

    @pl.loop(0, (N_ROW_BLOCKS + NS - 1) // NS)
    def _copy_out(k):
        blk = k * NS + sid

        @pl.when(blk < N_ROW_BLOCKS)
        def _():
            pltpu.sync_copy(
                agg.at[pl.ds(blk * ROW_BLK, ROW_BLK)],
                out_hbm.at[cid, pl.ds(blk * ROW_BLK, ROW_BLK)],
            )


@functools.cache
def _sc_aggregate():
    mesh = plsc.VectorSubcoreMesh(
        core_axis_name="c", subcore_axis_name="s",
        num_cores=NC, num_subcores=NS)
    return pl.kernel(
        _sc_body,
        out_type=jax.ShapeDtypeStruct((NC, N_NODES_C, D_C), jnp.float32),
        mesh=mesh,
        scratch_types=[
            pltpu.VMEM((2, CH), jnp.int32),
            pltpu.VMEM((2, CH), jnp.int32),
            pltpu.VMEM((2, CH), jnp.int32),
            pltpu.VMEM((CH, D_C), jnp.float32),
            pltpu.VMEM((CH, D_C), jnp.float32),
            pltpu.VMEM((CH, D_C), jnp.float32),
            pltpu.VMEM_SHARED((N_NODES_C, D_C), jnp.float32),
            pltpu.SemaphoreType.DMA,
            pltpu.SemaphoreType.DMA,
            pltpu.SemaphoreType.DMA,
            pltpu.SemaphoreType.DMA,
            pltpu.SemaphoreType.DMA,
            pltpu.SemaphoreType.DMA,
            pltpu.SemaphoreType.DMA,
            pltpu.SemaphoreType.DMA,
            pltpu.SemaphoreType.DMA,
        ],
    )


BLK = 2000


def _tc_body(agg_ref, node_ref, batch_ref, w1_ref, b1_ref, w2_ref, b2_ref,
             eps_ref, g_ref, bt_ref, out_ref, scale_ref):
    i = pl.program_id(0)

    @pl.when(i == 0)
    def _():
        bfull = batch_ref[...]
        giota = lax.broadcasted_iota(jnp.int32, (N_NODES_C, NUM_GRAPHS_C), 1)
        onehot = (bfull == giota).astype(jnp.float32)
        counts = jnp.sum(onehot, axis=0, keepdims=True)
        scale_ref[...] = lax.rsqrt(jnp.maximum(counts, 1.0))

    aggs = agg_ref[...]
    node = node_ref[...]
    x = aggs[0] + aggs[1] + (1.0 + eps_ref[0, 0]) * node
    h = jnp.dot(x, w1_ref[...], preferred_element_type=jnp.float32,
                precision=lax.Precision.HIGHEST) + b1_ref[...]
    h = jnp.maximum(h, 0.0)
    y = jnp.dot(h, w2_ref[...], preferred_element_type=jnp.float32,
                precision=lax.Precision.HIGHEST) + b2_ref[...]
    mu = jnp.mean(y, axis=-1, keepdims=True)
    var = jnp.mean((y - mu) ** 2, axis=-1, keepdims=True)
    y = (y - mu) * lax.rsqrt(var + 1e-5) * g_ref[...] + bt_ref[...]
    scale_g = scale_ref[...]
    bblk = batch_ref[pl.ds(i * BLK, BLK), :]
    biota = lax.broadcasted_iota(jnp.int32, (BLK, NUM_GRAPHS_C), 1)
    mask = (bblk == biota).astype(jnp.float32)
    scale = jnp.sum(mask * scale_g, axis=1, keepdims=True)
    y = jnp.maximum(y * scale, 0.0)
    out_ref[...] = y + node


_tc_mlp = pl.pallas_call(
    _tc_body,
    grid=(N_NODES_C // BLK,),
    in_specs=[
        pl.BlockSpec((NC, BLK, D_C), lambda i: (0, i, 0)),
        pl.BlockSpec((BLK, D_C), lambda i: (i, 0)),
        pl.BlockSpec((N_NODES_C, 1), lambda i: (0, 0)),
        pl.BlockSpec((D_C, 2 * D_C), lambda i: (0, 0)),
        pl.BlockSpec((1, 2 * D_C), lambda i: (0, 0)),
        pl.BlockSpec((2 * D_C, D_C), lambda i: (0, 0)),
        pl.BlockSpec((1, D_C), lambda i: (0, 0)),
        pl.BlockSpec((1, 1), lambda i: (0, 0)),
        pl.BlockSpec((1, D_C), lambda i: (0, 0)),
        pl.BlockSpec((1, D_C), lambda i: (0, 0)),
    ],
    out_specs=pl.BlockSpec((BLK, D_C), lambda i: (i, 0)),
    out_shape=jax.ShapeDtypeStruct((N_NODES_C, D_C), jnp.float32),
    scratch_shapes=[pltpu.VMEM((1, NUM_GRAPHS_C), jnp.float32)],
)


def kernel(node_hidden, edge_index, edge_hidden, batch, W1, b1, W2, b2, eps,
           ln_gamma, ln_beta):
    idx2 = edge_index.astype(jnp.int32)
    edge3 = edge_hidden.reshape(N_CHUNKS, CH, D_C)
    agg2 = _sc_aggregate()(idx2, edge3, node_hidden)
    out = _tc_mlp(
        agg2,
        node_hidden,
        batch.astype(jnp.int32).reshape(N_NODES_C, 1),
        W1,
        b1.reshape(1, 2 * D_C),
        W2,
        b2.reshape(1, D_C),
        eps.reshape(1, 1),
        ln_gamma.reshape(1, D_C),
        ln_beta.reshape(1, D_C),
    )
    return out
